# Initial kernel scaffold; baseline (speedup 1.0000x reference)
import jax
import jax.numpy as jnp
from jax.experimental import pallas as pl


def kernel(X, C, D, node_h, edge_h, edge_idx, mask_i, mask_ij, permute_idx, W_D, W_X, Wm, bm, We, be, Wd1, bd1, Wd2, bd2, Wf1, bf1, Wf2, bf2):
    raise NotImplementedError("write your pallas kernel here")


# speedup vs baseline: 1.8809x; 1.8809x over previous
import functools

import jax
import jax.numpy as jnp
from jax import lax
from jax.experimental import pallas as pl
from jax.experimental.pallas import tpu as pltpu
from jax.experimental.pallas import tpu_sc as plsc

_pallas_call = pl.pallas_call

_NC, _NS = 2, 16
_NW = _NC * _NS
_LANES = 16

_RANK_BITS = 14


def _softplus(x):
    return jnp.maximum(x, 0.0) + jnp.log1p(jnp.exp(-jnp.abs(x)))


def _layernorm(x):
    mu = jnp.mean(x, axis=-1, keepdims=True)
    xc = x - mu
    var = jnp.mean(xc * xc, axis=-1, keepdims=True)
    return xc * lax.rsqrt(var + 1e-5)


def _dot(a, b):
    return jnp.dot(a, b, preferred_element_type=jnp.float32,
                   precision=lax.Precision.HIGHEST)


def _rep_mat(rows, nb, k):
    rg = lax.broadcasted_iota(jnp.int32, (rows, nb), 0) // k
    cc = lax.broadcasted_iota(jnp.int32, (rows, nb), 1)
    return (rg == cc).astype(jnp.float32)


def _sel_mat(nb, rows, k):
    rg = lax.broadcasted_iota(jnp.int32, (nb, rows), 1) // k
    cc = lax.broadcasted_iota(jnp.int32, (nb, rows), 0)
    return (rg == cc).astype(jnp.float32)


def _sc_gather_rows(table, idx):
    V, D = table.shape
    Bp = idx.shape[0]
    bpw = Bp // _NW
    ch = 128
    nchunks = bpw // ch
    mesh = plsc.VectorSubcoreMesh(core_axis_name="c", subcore_axis_name="s")

    @functools.partial(
        pl.kernel,
        mesh=mesh,
        out_type=jax.ShapeDtypeStruct((Bp, D), table.dtype),
        scratch_types=[
            pltpu.VMEM((bpw,), jnp.int32),
            pltpu.VMEM((ch, D), table.dtype),
            pltpu.SemaphoreType.DMA,
        ],
        compiler_params=pltpu.CompilerParams(use_tc_tiling_on_sc=False),
    )
    def k(table_hbm, idx_hbm, out_hbm, idx_v, rows_v, sem):
        wid = lax.axis_index("s") * _NC + lax.axis_index("c")
        base = wid * bpw
        pltpu.sync_copy(idx_hbm.at[pl.ds(base, bpw)], idx_v)

        def body(c, carry):
            pltpu.async_copy(
                table_hbm.at[idx_v.at[pl.ds(c * ch, ch)]], rows_v, sem
            ).wait()
            pltpu.sync_copy(rows_v, out_hbm.at[pl.ds(base + c * ch, ch)])
            return carry

        lax.fori_loop(0, nchunks, body, 0)

    return k(table, idx)


def _init_body(x_ref, nh_ref, wx_ref, w1_ref, w2_ref, b1_ref,
               out_nh, out_a, out_p):
    n0 = nh_ref[...] + x_ref[...] * wx_ref[...]
    out_nh[...] = n0
    out_a[...] = _dot(n0, w1_ref[...]) + b1_ref[...]
    out_p[...] = _dot(n0, w2_ref[...])


def _prologue_body(nb, rows, k, eh_ref, pk_ref, rank_ref, mij_ref, wd_ref,
                   out_eh, out_mf):
    pk = pk_ref[...][:, 0:1]
    dj = lax.shift_right_logical(pk, _RANK_BITS)
    rj = (pk & ((1 << _RANK_BITS) - 1)).astype(jnp.float32)
    ri = _dot(_rep_mat(rows, nb, k), rank_ref[...])
    mar = (rj < ri).astype(jnp.float32)
    ncls = wd_ref.shape[0]
    oh = (dj == lax.broadcasted_iota(jnp.int32, (rows, ncls), 1)
          ).astype(jnp.float32)
    out_eh[...] = eh_ref[...] + _dot(oh, wd_ref[...]) * mar
    out_mf[...] = mij_ref[...] * mar


def _node_body(nb, rows, k, has_next, *refs):
    if has_next:
        (nh_ref, a_ref, pj_ref, eh_ref, mf_ref, mi_ref, w3_ref,
         we1_ref, we2_ref, be_ref, wn1_ref, wn2_ref, bn_ref,
         out_nh, out_a2, out_p2, out_an, out_pn) = refs
    else:
        (nh_ref, a_ref, pj_ref, eh_ref, mf_ref, mi_ref, w3_ref,
         we1_ref, we2_ref, be_ref,
         out_nh, out_a2, out_p2) = refs
    rep = _rep_mat(rows, nb, k)
    x = _dot(eh_ref[...], w3_ref[...]) + pj_ref[...] + _dot(rep, a_ref[...])
    msg = _softplus(x) * mf_ref[...]
    agg = _dot(_sel_mat(nb, rows, k), msg) * (1.0 / k)
    nn = _layernorm(nh_ref[...] + agg) * mi_ref[...]
    out_nh[...] = nn
    out_a2[...] = _dot(nn, we1_ref[...]) + be_ref[...]
    out_p2[...] = _dot(nn, we2_ref[...])
    if has_next:
        out_an[...] = _dot(nn, wn1_ref[...]) + bn_ref[...]
        out_pn[...] = _dot(nn, wn2_ref[...])


def _edge_body(nb, rows, k, eh_ref, pj_ref, a_ref, mf_ref, w3_ref, out_eh):
    rep = _rep_mat(rows, nb, k)
    x = _dot(eh_ref[...], w3_ref[...]) + pj_ref[...] + _dot(rep, a_ref[...])
    h = eh_ref[...] + _softplus(x)
    out_eh[...] = _layernorm(h) * mf_ref[...]


def _decoder_body(nh_ref, d_ref, mi_ref, wd1_ref, bd1_ref, wd2_ref, bd2_ref,
                  wf1_ref, bf1_ref, wf2_ref, bf2_ref, out_lp, out_lf):
    h = nh_ref[...]
    hd = jnp.maximum(_dot(h, wd1_ref[...]) + bd1_ref[...], 0.0)
    lg = _dot(hd, wd2_ref[...]) + bd2_ref[...]
    m = jnp.max(lg, axis=-1, keepdims=True)
    lse = jnp.log(jnp.sum(jnp.exp(lg - m), axis=-1, keepdims=True)) + m
    ncls = lg.shape[-1]
    oh = (d_ref[...] == lax.broadcasted_iota(jnp.int32, (lg.shape[0], ncls), 1)
          ).astype(jnp.float32)
    pick = jnp.sum(lg * oh, axis=-1, keepdims=True)
    out_lp[...] = (pick - lse) * mi_ref[...]
    hf = jnp.maximum(_dot(h, wf1_ref[...]) + bf1_ref[...], 0.0)
    out_lf[...] = _dot(hf, wf2_ref[...]) + bf2_ref[...]


def kernel(X, C, D, node_h, edge_h, edge_idx, mask_i, mask_ij, permute_idx,
           W_D, W_X, Wm, bm, We, be, Wd1, bd1, Wd2, bd2, Wf1, bf1, Wf2, bf2):
    B, N, K = edge_idx.shape
    dn = node_h.shape[-1]
    de = edge_h.shape[-1]
    L = Wm.shape[0]
    E = N * K

    rank = jnp.argsort(permute_idx[0]).astype(jnp.int32)
    d_flat = D.reshape(N)
    packed = d_flat * (1 << _RANK_BITS) + rank
    idx_flat = edge_idx.reshape(E)
    Ep = ((E + _NW * 128 - 1) // (_NW * 128)) * (_NW * 128)
    idx_pad = jnp.concatenate(
        [idx_flat, jnp.zeros((Ep - E,), jnp.int32)])
    x_col = X.reshape(N, 1)
    rank_col = rank.reshape(N, 1).astype(jnp.float32)
    d_col = d_flat.reshape(N, 1)
    mi_col = mask_i.reshape(N, 1)
    mij_col = mask_ij.reshape(E, 1)
    eh_flat = edge_h.reshape(E, de)
    nh0 = node_h.reshape(N, dn)
    bm_r = bm.reshape(L, 1, dn)
    be_r = be.reshape(L, 1, de)

    nb = 40
    rows = nb * K
    gN = N // nb
    f32 = jnp.float32

    def spec(bs, ndim=2):
        return pl.BlockSpec(bs, lambda i: (i,) + (0,) * (len(bs) - 1))

    def wspec(shape):
        return pl.BlockSpec(shape, lambda i: (0,) * len(shape))

    sds = jax.ShapeDtypeStruct

    packed_tab = jnp.broadcast_to(packed[:, None], (N, 16))
    pk_rows = _sc_gather_rows(packed_tab, idx_pad)

    nbi = 400
    node0, a0, p0 = _pallas_call(
        _init_body,
        grid=(N // nbi,),
        in_specs=[spec((nbi, 1)), spec((nbi, dn)), wspec((1, dn)),
                  wspec((dn, dn)), wspec((dn, dn)), wspec((1, dn))],
        out_specs=[spec((nbi, dn))] * 3,
        out_shape=[sds((N, dn), f32)] * 3,
    )(x_col, nh0, W_X, Wm[0, :dn], Wm[0, dn:2 * dn], bm_r[0])

    eh_eff, mask_flat = _pallas_call(
        functools.partial(_prologue_body, nb, rows, K),
        grid=(gN,),
        in_specs=[spec((rows, de)), spec((rows, 16)), spec((nb, 1)),
                  spec((rows, 1)), wspec(W_D.shape)],
        out_specs=[spec((rows, de)), spec((rows, 1))],
        out_shape=[sds((E, de), f32), sds((E, 1), f32)],
    )(eh_flat, pk_rows, rank_col, mij_col, W_D)

    nh, a, p = node0, a0, p0
    eh = eh_eff
    for l in range(L):
        has_next = l + 1 < L
        pj = _sc_gather_rows(p, idx_pad)
        ins = [nh, a, pj, eh, mask_flat, mi_col,
               Wm[l, 2 * dn:], We[l, :dn], We[l, dn:2 * dn], be_r[l]]
        in_specs = [spec((nb, dn)), spec((nb, dn)), spec((rows, dn)),
                    spec((rows, de)), spec((rows, 1)), spec((nb, 1)),
                    wspec((de, dn)), wspec((dn, de)), wspec((dn, de)),
                    wspec((1, de))]
        n_out = 3
        if has_next:
            ins += [Wm[l + 1, :dn], Wm[l + 1, dn:2 * dn], bm_r[l + 1]]
            in_specs += [wspec((dn, dn)), wspec((dn, dn)), wspec((1, dn))]
            n_out = 5
        outs = _pallas_call(
            functools.partial(_node_body, nb, rows, K, has_next),
            grid=(gN,),
            in_specs=in_specs,
            out_specs=[spec((nb, dn))] * n_out,
            out_shape=[sds((N, dn), f32)] * n_out,
        )(*ins)
        if has_next:
            nh, a2, p2, a, p = outs
        else:
            nh, a2, p2 = outs
        p2j = _sc_gather_rows(p2, idx_pad)
        eh = _pallas_call(
            functools.partial(_edge_body, nb, rows, K),
            grid=(gN,),
            in_specs=[spec((rows, de)), spec((rows, de)), spec((nb, de)),
                      spec((rows, 1)), wspec((de, de))],
            out_specs=spec((rows, de)),
            out_shape=sds((E, de), f32),
        )(eh, p2j, a2, mask_flat, We[l, 2 * dn:])

    dh = Wd1.shape[1]
    ncls = Wd2.shape[1]
    nfb = Wf2.shape[1]
    logp, logits_field = _pallas_call(
        _decoder_body,
        grid=(N // nbi,),
        in_specs=[spec((nbi, dn)), spec((nbi, 1)), spec((nbi, 1)),
                  wspec((dn, dh)), wspec((1, dh)), wspec((dh, ncls)),
                  wspec((1, ncls)), wspec((dn, dh)), wspec((1, dh)),
                  wspec((dh, nfb)), wspec((1, nfb))],
        out_specs=[spec((nbi, 1)), spec((nbi, nfb))],
        out_shape=[sds((N, 1), f32), sds((N, nfb), f32)],
    )(nh, d_col, mi_col, Wd1, bd1.reshape(1, dh), Wd2, bd2.reshape(1, ncls),
      Wf1, bf1.reshape(1, dh), Wf2, bf2.reshape(1, nfb))

    return (logp.reshape(B, N),
            logits_field.reshape(B, N, nfb),
            nh.reshape(B, N, dn),
            eh.reshape(B, N, K, de))
